# baseline (speedup 1.0000x reference)
import functools

import jax
import jax.numpy as jnp
import numpy as np
from jax import lax
from jax.experimental import pallas as pl
from jax.experimental.pallas import tpu as pltpu
from jax.experimental.pallas import tpu_sc as plsc

D_MODEL = 64
MAXLEN = 200
VOCAB = 5
BATCH = 4096
ROWS = BATCH * MAXLEN
NC, NS = 2, 16
NW = NC * NS

B_SC = 1024
ROWS_SC = B_SC * MAXLEN
ROWS_TC = ROWS - ROWS_SC

RPW = ROWS_SC // NW
CHUNK = 400
NCHUNK = RPW // CHUNK
JV = CHUNK // 16
NGATHER = 5
GR = CHUNK // NGATHER
L_PER = 13

BM2 = 3200
NBLKF = ROWS // BM2
NSCB = ROWS_SC // BM2


def _positional_encoding() -> np.ndarray:
    pos = np.arange(MAXLEN)[:, None]
    i = np.arange(D_MODEL)[None, :]
    rates = 1 / np.power(10000, 2 * (i // 2) / np.float32(D_MODEL))
    angle = pos * rates
    angle[:, 0::2] = np.sin(angle[:, 0::2])
    angle[:, 1::2] = np.cos(angle[:, 1::2])
    return angle.astype(np.float32)


_PE = _positional_encoding()


def _sc_body(tok_hbm, table_hbm, pe_hbm, out_hbm,
             pe_v, tab_v, build_v, fused_sh, pos5_v, tok_v, idx_v, rows_v,
             tsem, gsem, wsem0, wsem1):
    s = lax.axis_index("s")
    c = lax.axis_index("c")
    wid = s * NC + c

    pltpu.sync_copy(table_hbm, tab_v)
    pltpu.sync_copy(pe_hbm, pe_v)
    for v in range(VOCAB):
        for k in range(D_MODEL // 16):
            tab_v[v, pl.ds(k * 16, 16)] = tab_v[v, pl.ds(k * 16, 16)] * 8.0
    l0 = s * L_PER
    for li in range(L_PER):
        l = l0 + li

        @pl.when(l < MAXLEN)
        def _build():
            for v in range(VOCAB):
                for k in range(D_MODEL // 16):
                    sl = pl.ds(k * 16, 16)
                    build_v[v, sl] = (tab_v[v, sl] + pe_v[l, sl]) * 0.125
            pltpu.sync_copy(build_v, fused_sh.at[pl.ds(l * VOCAB, VOCAB)])

    plsc.subcore_barrier()

    iota16 = lax.broadcasted_iota(jnp.int32, (16,), 0)
    for j in range(JV):
        pos5_v[pl.ds(j * 16, 16)] = lax.rem(j * 16 + iota16, MAXLEN) * VOCAB

    row_w = wid * RPW
    wsems = (wsem0, wsem1)

    pltpu.async_copy(tok_hbm.at[pl.ds(row_w, CHUNK)], tok_v.at[0], tsem)

    @pl.loop(0, NCHUNK, step=2)
    def _chunk2(g0):
        for p in range(2):
            g = g0 + p
            row0 = row_w + g * CHUNK
            pltpu.make_async_copy(
                tok_hbm.at[pl.ds(row0, CHUNK)], tok_v.at[p], tsem).wait()

            @pl.when(g + 1 < NCHUNK)
            def _prefetch():
                pltpu.async_copy(
                    tok_hbm.at[pl.ds(row0 + CHUNK, CHUNK)],
                    tok_v.at[1 - p], tsem)

            for j in range(JV):
                sl = pl.ds(j * 16, 16)
                idx_v[p, j // (GR // 16), pl.ds((j % (GR // 16)) * 16, 16)] = (
                    pos5_v[sl] + tok_v[p, sl])

            @pl.when(g0 > 0)
            def _drain():
                pltpu.make_async_copy(
                    rows_v.at[p],
                    out_hbm.at[pl.ds(row0 - 2 * CHUNK, CHUNK)],
                    wsems[p]).wait()

            copies = []
            for r in range(NGATHER):
                copies.append(pltpu.async_copy(
                    fused_sh.at[idx_v.at[p, r]],
                    rows_v.at[p, pl.ds(r * GR, GR)], gsem))
            for cp in copies:
                cp.wait()
            pltpu.async_copy(
                rows_v.at[p], out_hbm.at[pl.ds(row0, CHUNK)], wsems[p])

    for p in range(2):
        row0 = row_w + (NCHUNK - 2 + p) * CHUNK
        pltpu.make_async_copy(
            rows_v.at[p], out_hbm.at[pl.ds(row0, CHUNK)], wsems[p]).wait()


def _tc_body(sc_ref, tok_ref, table_ref, pe2_ref, out_ref):
    i = pl.program_id(0)

    @pl.when(i < NSCB)
    def _passthrough():
        out_ref[...] = sc_ref[...]

    @pl.when(i >= NSCB)
    def _compute():
        onehot = (tok_ref[...] == lax.broadcasted_iota(
            jnp.int32, (BM2, 8), 1)).astype(jnp.bfloat16)
        dims = (((1,), (0,)), ((), ()))
        gathered = lax.dot_general(
            onehot, table_ref[0], dims,
            preferred_element_type=jnp.float32)
        gathered += lax.dot_general(
            onehot, table_ref[1], dims,
            preferred_element_type=jnp.float32)
        gathered += lax.dot_general(
            onehot, table_ref[2], dims,
            preferred_element_type=jnp.float32)
        out_ref[...] = gathered + pe2_ref[...]


def _tc_embed(sc_out, tok_col, table8, pe2):
    return pl.pallas_call(
        _tc_body,
        grid=(NBLKF,),
        in_specs=[
            pl.BlockSpec((BM2, D_MODEL),
                         lambda i: (jnp.minimum(i, NSCB - 1), 0)),
            pl.BlockSpec((BM2, 1), lambda i: (i, 0)),
            pl.BlockSpec((3, 8, D_MODEL), lambda i: (0, 0, 0)),
            pl.BlockSpec((BM2, D_MODEL), lambda i: (0, 0)),
        ],
        out_specs=pl.BlockSpec((BM2, D_MODEL), lambda i: (i, 0)),
        out_shape=jax.ShapeDtypeStruct((ROWS, D_MODEL), jnp.float32),
        compiler_params=pltpu.CompilerParams(
            dimension_semantics=("arbitrary",)),
    )(sc_out, tok_col, table8, pe2)


@functools.partial(jax.jit, static_argnames=())
def _embed(tok_flat, emb_table, pe, pe2):
    mesh = plsc.VectorSubcoreMesh(core_axis_name="c", subcore_axis_name="s",
                                  num_cores=NC, num_subcores=NS)
    sc_out = pl.kernel(
        _sc_body,
        out_type=jax.ShapeDtypeStruct((ROWS_SC, D_MODEL), jnp.float32),
        mesh=mesh,
        scratch_types=[
            pltpu.VMEM((MAXLEN, D_MODEL), jnp.float32),
            pltpu.VMEM((VOCAB, D_MODEL), jnp.float32),
            pltpu.VMEM((VOCAB, D_MODEL), jnp.float32),
            pltpu.VMEM_SHARED((MAXLEN * VOCAB, D_MODEL), jnp.float32),
            pltpu.VMEM((CHUNK,), jnp.int32),
            pltpu.VMEM((2, CHUNK), jnp.int32),
            pltpu.VMEM((2, NGATHER, GR), jnp.int32),
            pltpu.VMEM((2, CHUNK, D_MODEL), jnp.float32),
            pltpu.SemaphoreType.DMA,
            pltpu.SemaphoreType.DMA,
            pltpu.SemaphoreType.DMA,
            pltpu.SemaphoreType.DMA,
        ],
        compiler_params=pltpu.CompilerParams(use_tc_tiling_on_sc=False),
    )(tok_flat, emb_table, pe)
    tok_col = tok_flat.reshape(ROWS, 1)
    table8 = jnp.concatenate(
        [emb_table, jnp.zeros((8 - VOCAB, D_MODEL), jnp.float32)], axis=0)
    t_hi = table8.astype(jnp.bfloat16)
    r1 = table8 - t_hi.astype(jnp.float32)
    t_mid = r1.astype(jnp.bfloat16)
    t_lo = (r1 - t_mid.astype(jnp.float32)).astype(jnp.bfloat16)
    table3 = jnp.stack([t_hi, t_mid, t_lo])
    out = _tc_embed(sc_out, tok_col, table3, pe2)
    return out.reshape(BATCH, MAXLEN, D_MODEL)


_PE2 = np.tile(_PE, (BM2 // MAXLEN, 1)) * np.float32(0.125)


def kernel(rnatok, emb_table):
    pe = jnp.asarray(_PE)
    pe2 = jnp.asarray(_PE2)
    return _embed(rnatok.reshape(-1), emb_table, pe, pe2)


# speedup vs baseline: 1.4429x; 1.4429x over previous
import functools

import jax
import jax.numpy as jnp
import numpy as np
from jax import lax
from jax.experimental import pallas as pl
from jax.experimental.pallas import tpu as pltpu
from jax.experimental.pallas import tpu_sc as plsc

D_MODEL = 64
MAXLEN = 200
VOCAB = 5
BATCH = 4096
ROWS = BATCH * MAXLEN
NC, NS = 2, 16
NW = NC * NS

B_SC = 1024
ROWS_SC = B_SC * MAXLEN
ROWS_TC = ROWS - ROWS_SC

RPW = ROWS_SC // NW
CHUNK = 400
NCHUNK = RPW // CHUNK
JV = CHUNK // 16
NGATHER = 5
GR = CHUNK // NGATHER
L_PER = 13

BM2 = 3200
NBLKF = ROWS // BM2
NSCB = ROWS_SC // BM2


def _positional_encoding() -> np.ndarray:
    pos = np.arange(MAXLEN)[:, None]
    i = np.arange(D_MODEL)[None, :]
    rates = 1 / np.power(10000, 2 * (i // 2) / np.float32(D_MODEL))
    angle = pos * rates
    angle[:, 0::2] = np.sin(angle[:, 0::2])
    angle[:, 1::2] = np.cos(angle[:, 1::2])
    return angle.astype(np.float32)


_PE = _positional_encoding()


def _sc_body(tok_hbm, table_hbm, pe_hbm, out_hbm,
             pe_v, tab_v, build_v, fused_sh, pos5_v, tok_v, idx_v, rows_v,
             tsem, gsem, wsem0, wsem1):
    s = lax.axis_index("s")
    c = lax.axis_index("c")
    wid = s * NC + c

    pltpu.sync_copy(table_hbm, tab_v)
    pltpu.sync_copy(pe_hbm, pe_v)
    for v in range(VOCAB):
        for k in range(D_MODEL // 16):
            tab_v[v, pl.ds(k * 16, 16)] = tab_v[v, pl.ds(k * 16, 16)] * 8.0
    l0 = s * L_PER
    for li in range(L_PER):
        l = l0 + li

        @pl.when(l < MAXLEN)
        def _build():
            for v in range(VOCAB):
                for k in range(D_MODEL // 16):
                    sl = pl.ds(k * 16, 16)
                    build_v[v, sl] = (tab_v[v, sl] + pe_v[l, sl]) * 0.125
            pltpu.sync_copy(build_v, fused_sh.at[pl.ds(l * VOCAB, VOCAB)])

    plsc.subcore_barrier()

    iota16 = lax.broadcasted_iota(jnp.int32, (16,), 0)
    for j in range(JV):
        pos5_v[pl.ds(j * 16, 16)] = lax.rem(j * 16 + iota16, MAXLEN) * VOCAB

    row_w = wid * RPW
    wsems = (wsem0, wsem1)

    pltpu.async_copy(tok_hbm.at[pl.ds(row_w, CHUNK)], tok_v.at[0], tsem)

    @pl.loop(0, NCHUNK, step=2)
    def _chunk2(g0):
        for p in range(2):
            g = g0 + p
            row0 = row_w + g * CHUNK
            pltpu.make_async_copy(
                tok_hbm.at[pl.ds(row0, CHUNK)], tok_v.at[p], tsem).wait()

            @pl.when(g + 1 < NCHUNK)
            def _prefetch():
                pltpu.async_copy(
                    tok_hbm.at[pl.ds(row0 + CHUNK, CHUNK)],
                    tok_v.at[1 - p], tsem)

            for j in range(JV):
                sl = pl.ds(j * 16, 16)
                idx_v[p, j // (GR // 16), pl.ds((j % (GR // 16)) * 16, 16)] = (
                    pos5_v[sl] + tok_v[p, sl])

            @pl.when(g0 > 0)
            def _drain():
                pltpu.make_async_copy(
                    rows_v.at[p],
                    out_hbm.at[pl.ds(row0 - 2 * CHUNK, CHUNK)],
                    wsems[p]).wait()

            copies = []
            for r in range(NGATHER):
                copies.append(pltpu.async_copy(
                    fused_sh.at[idx_v.at[p, r]],
                    rows_v.at[p, pl.ds(r * GR, GR)], gsem))
            for cp in copies:
                cp.wait()
            pltpu.async_copy(
                rows_v.at[p], out_hbm.at[pl.ds(row0, CHUNK)], wsems[p])

    for p in range(2):
        row0 = row_w + (NCHUNK - 2 + p) * CHUNK
        pltpu.make_async_copy(
            rows_v.at[p], out_hbm.at[pl.ds(row0, CHUNK)], wsems[p]).wait()


def _tc_body(sc_ref, tok_ref, table_ref, pe2_ref, out_ref):
    i = pl.program_id(0)

    @pl.when(i < NSCB)
    def _passthrough():
        out_ref[...] = sc_ref[...]

    @pl.when(i >= NSCB)
    def _compute():
        oht = (tok_ref[0][None] == lax.broadcasted_iota(
            jnp.int32, (8, BM2 // 128, 128), 0)
        ).astype(jnp.bfloat16).reshape(8, BM2)
        dims = (((0,), (0,)), ((), ()))
        gathered = lax.dot_general(
            oht, table_ref[0], dims,
            preferred_element_type=jnp.float32)
        gathered += lax.dot_general(
            oht, table_ref[1], dims,
            preferred_element_type=jnp.float32)
        gathered += lax.dot_general(
            oht, table_ref[2], dims,
            preferred_element_type=jnp.float32)
        out_ref[...] = gathered + pe2_ref[...]


def _tc_embed(sc_out, tok_mat, table8, pe2):
    return pl.pallas_call(
        _tc_body,
        grid=(NBLKF,),
        in_specs=[
            pl.BlockSpec((BM2, D_MODEL),
                         lambda i: (jnp.minimum(i, NSCB - 1), 0)),
            pl.BlockSpec((1, BM2 // 128, 128), lambda i: (i, 0, 0)),
            pl.BlockSpec((3, 8, D_MODEL), lambda i: (0, 0, 0)),
            pl.BlockSpec((BM2, D_MODEL), lambda i: (0, 0)),
        ],
        out_specs=pl.BlockSpec((BM2, D_MODEL), lambda i: (i, 0)),
        out_shape=jax.ShapeDtypeStruct((ROWS, D_MODEL), jnp.float32),
        compiler_params=pltpu.CompilerParams(
            dimension_semantics=("arbitrary",)),
    )(sc_out, tok_mat, table8, pe2)


@functools.partial(jax.jit, static_argnames=())
def _embed(tok_flat, emb_table, pe, pe2):
    mesh = plsc.VectorSubcoreMesh(core_axis_name="c", subcore_axis_name="s",
                                  num_cores=NC, num_subcores=NS)
    sc_out = pl.kernel(
        _sc_body,
        out_type=jax.ShapeDtypeStruct((ROWS_SC, D_MODEL), jnp.float32),
        mesh=mesh,
        scratch_types=[
            pltpu.VMEM((MAXLEN, D_MODEL), jnp.float32),
            pltpu.VMEM((VOCAB, D_MODEL), jnp.float32),
            pltpu.VMEM((VOCAB, D_MODEL), jnp.float32),
            pltpu.VMEM_SHARED((MAXLEN * VOCAB, D_MODEL), jnp.float32),
            pltpu.VMEM((CHUNK,), jnp.int32),
            pltpu.VMEM((2, CHUNK), jnp.int32),
            pltpu.VMEM((2, NGATHER, GR), jnp.int32),
            pltpu.VMEM((2, CHUNK, D_MODEL), jnp.float32),
            pltpu.SemaphoreType.DMA,
            pltpu.SemaphoreType.DMA,
            pltpu.SemaphoreType.DMA,
            pltpu.SemaphoreType.DMA,
        ],
        compiler_params=pltpu.CompilerParams(use_tc_tiling_on_sc=False),
    )(tok_flat, emb_table, pe)
    tok_mat = tok_flat.reshape(NBLKF, BM2 // 128, 128)
    table8 = jnp.concatenate(
        [emb_table, jnp.zeros((8 - VOCAB, D_MODEL), jnp.float32)], axis=0)
    t_hi = table8.astype(jnp.bfloat16)
    r1 = table8 - t_hi.astype(jnp.float32)
    t_mid = r1.astype(jnp.bfloat16)
    t_lo = (r1 - t_mid.astype(jnp.float32)).astype(jnp.bfloat16)
    table3 = jnp.stack([t_hi, t_mid, t_lo])
    out = _tc_embed(sc_out, tok_mat, table3, pe2)
    return out.reshape(BATCH, MAXLEN, D_MODEL)


_PE2 = np.tile(_PE, (BM2 // MAXLEN, 1)) * np.float32(0.125)


def kernel(rnatok, emb_table):
    pe = jnp.asarray(_PE)
    pe2 = jnp.asarray(_PE2)
    return _embed(rnatok.reshape(-1), emb_table, pe, pe2)
